# Initial kernel scaffold; baseline (speedup 1.0000x reference)
import jax
import jax.numpy as jnp
from jax.experimental import pallas as pl


def kernel(x_type, x_tok, x_small, batch, W1, b1, W2, b2, W3, b3):
    raise NotImplementedError("write your pallas kernel here")


# speedup vs baseline: 15.8367x; 15.8367x over previous
import functools

import jax
import jax.numpy as jnp
from jax import lax
from jax.experimental import pallas as pl
from jax.experimental.pallas import tpu as pltpu
from jax.experimental.pallas import tpu_sc as plsc

_N = 160000
_G = 512
_NT = 128
_NK = 129
_NW = 32
_CHUNK = _N // _NW
_SBUF = 5008
_ITER = _SBUF // 16
_HT = _G * _NT
_HK = _G * _NK
_NEG = float("-inf")


def _take(x, idx):
  return x.at[idx].get(mode="promise_in_bounds")


def _sc_pool_body(bat_h, typ_h, tok_h, xs0_h, xs1_h,
                  out_t, out_k, out_s, out_m,
                  b_v, t_v, k_v, s0_v, s1_v, hist, sacc, macc):
  wid = lax.axis_index("s") * 2 + lax.axis_index("c")
  base = wid * _CHUNK
  pltpu.sync_copy(bat_h.at[pl.ds(base, _CHUNK)], b_v.at[pl.ds(0, _CHUNK)])
  pltpu.sync_copy(typ_h.at[pl.ds(base, _CHUNK)], t_v.at[pl.ds(0, _CHUNK)])
  pltpu.sync_copy(tok_h.at[pl.ds(base, _CHUNK)], k_v.at[pl.ds(0, _CHUNK)])
  pltpu.sync_copy(xs0_h.at[pl.ds(base, _CHUNK)], s0_v.at[pl.ds(0, _CHUNK)])
  pltpu.sync_copy(xs1_h.at[pl.ds(base, _CHUNK)], s1_v.at[pl.ds(0, _CHUNK)])

  lanes = lax.iota(jnp.int32, 16)
  zero16 = jnp.zeros((16,), jnp.float32)
  ninf16 = jnp.full((16,), _NEG, jnp.float32)

  def _zero_hist(j, c):
    hist[pl.ds(j * 16, 16)] = zero16
    return c

  lax.fori_loop(0, _HK // 16, _zero_hist, 0)

  def _init_small(j, c):
    sacc[pl.ds(j * 16, 16)] = zero16
    return c

  lax.fori_loop(0, (_G * 4) // 16, _init_small, 0)

  def _init_max(j, c):
    macc[pl.ds(j * 16, 16)] = ninf16
    return c

  lax.fori_loop(0, (_G * 2) // 16, _init_max, 0)

  def _pass_a(i, c):
    off = i * 16
    valid = (off + lanes) < _CHUNK
    b = jnp.where(valid, b_v[pl.ds(off, 16)], -1)
    t = t_v[pl.ds(off, 16)]
    idx = b * _NT + t
    cnt, lastm = plsc.scan_count(idx, mask=valid)
    plsc.addupdate_scatter(hist, [idx], cnt.astype(jnp.float32), mask=lastm)
    return c

  lax.fori_loop(0, _ITER, _pass_a, 0)
  pltpu.sync_copy(hist.at[pl.ds(0, _HT)], out_t.at[wid])
  lax.fori_loop(0, _HK // 16, _zero_hist, 0)

  def _pass_b(i, c):
    off = i * 16
    valid = (off + lanes) < _CHUNK
    b = jnp.where(valid, b_v[pl.ds(off, 16)], -1)
    k = jnp.clip(k_v[pl.ds(off, 16)], 0, _NK - 1)
    idx = b * _NK + k
    cnt, lastm = plsc.scan_count(idx, mask=valid)
    plsc.addupdate_scatter(hist, [idx], cnt.astype(jnp.float32), mask=lastm)

    v0 = jnp.where(valid, s0_v[pl.ds(off, 16)], 0.0)
    v1 = jnp.where(valid, s1_v[pl.ds(off, 16)], 0.0)
    s0 = v0
    s1 = v1
    q0 = v0 * v0
    q1 = v1 * v1
    m0 = jnp.where(valid, v0, _NEG)
    m1 = jnp.where(valid, v1, _NEG)
    for d in (1, 2, 4, 8):
      sidx = jnp.maximum(lanes - d, 0)
      bd = _take(b, sidx)
      same = (bd == b) & (lanes >= d)
      s0 = s0 + jnp.where(same, _take(s0, sidx), 0.0)
      s1 = s1 + jnp.where(same, _take(s1, sidx), 0.0)
      q0 = q0 + jnp.where(same, _take(q0, sidx), 0.0)
      q1 = q1 + jnp.where(same, _take(q1, sidx), 0.0)
      m0 = jnp.maximum(m0, jnp.where(same, _take(m0, sidx), _NEG))
      m1 = jnp.maximum(m1, jnp.where(same, _take(m1, sidx), _NEG))
    nb = _take(b, jnp.minimum(lanes + 1, 15))
    lastseg = ((b != nb) | (lanes == 15)) & valid
    plsc.addupdate_scatter(sacc, [b * 4], s0, mask=lastseg)
    plsc.addupdate_scatter(sacc, [b * 4 + 1], s1, mask=lastseg)
    plsc.addupdate_scatter(sacc, [b * 4 + 2], q0, mask=lastseg)
    plsc.addupdate_scatter(sacc, [b * 4 + 3], q1, mask=lastseg)
    cur0 = plsc.load_gather(macc, [b * 2], mask=lastseg)
    plsc.store_scatter(macc, [b * 2], jnp.maximum(cur0, m0), mask=lastseg)
    cur1 = plsc.load_gather(macc, [b * 2 + 1], mask=lastseg)
    plsc.store_scatter(macc, [b * 2 + 1], jnp.maximum(cur1, m1), mask=lastseg)
    return c

  lax.fori_loop(0, _ITER, _pass_b, 0)
  pltpu.sync_copy(hist, out_k.at[wid])
  pltpu.sync_copy(sacc, out_s.at[wid])
  pltpu.sync_copy(macc, out_m.at[wid])


_sc_pool = functools.partial(
    pl.kernel,
    out_type=[
        jax.ShapeDtypeStruct((_NW, _HT), jnp.float32),
        jax.ShapeDtypeStruct((_NW, _HK), jnp.float32),
        jax.ShapeDtypeStruct((_NW, _G * 4), jnp.float32),
        jax.ShapeDtypeStruct((_NW, _G * 2), jnp.float32),
    ],
    mesh=plsc.VectorSubcoreMesh(core_axis_name="c", subcore_axis_name="s"),
    compiler_params=pltpu.CompilerParams(needs_layout_passes=False),
    scratch_types=[
        pltpu.VMEM((_SBUF,), jnp.int32),
        pltpu.VMEM((_SBUF,), jnp.int32),
        pltpu.VMEM((_SBUF,), jnp.int32),
        pltpu.VMEM((_SBUF,), jnp.float32),
        pltpu.VMEM((_SBUF,), jnp.float32),
        pltpu.VMEM((_HK,), jnp.float32),
        pltpu.VMEM((_G * 4,), jnp.float32),
        pltpu.VMEM((_G * 2,), jnp.float32),
    ],
)(_sc_pool_body)


def _leaky(v):
  return jnp.where(v > 0, v, 0.01 * v)


def _tc_mlp_body(ht, hk, hs, hm, w1, b1, w2, b2, w3, b3, out,
                 acc_t, acc_k, acc_s, acc_m):
  i = pl.program_id(0)

  @pl.when(i == 0)
  def _():
    acc_t[...] = jnp.zeros_like(acc_t)
    acc_k[...] = jnp.zeros_like(acc_k)
    acc_s[...] = jnp.zeros_like(acc_s)
    acc_m[...] = jnp.full_like(acc_m, _NEG)

  acc_t[...] += ht[0]
  acc_k[...] += hk[0]
  acc_s[...] += hs[0]
  acc_m[...] = jnp.maximum(acc_m[...], hm[0])

  @pl.when(i == _NW - 1)
  def _():
    at = acc_t[...]
    ak = acc_k[...]
    sm = acc_s[...]
    am = acc_m[...]
    cnt = jnp.sum(at, axis=1, keepdims=True)
    cntc = jnp.maximum(cnt, 1.0)
    mt = at / cntc
    mk = ak / cntc
    ms = sm[:, 0:2] / cntc
    qs = sm[:, 2:4] / cntc
    empty = cnt <= 0.0
    xt = jnp.where(empty, _NEG, (at > 0).astype(jnp.float32))
    xk = jnp.where(empty, _NEG, (ak > 0).astype(jnp.float32))
    st = jnp.sqrt(jnp.clip(mt - mt * mt, 0.0, None) + 1e-8)
    sk = jnp.sqrt(jnp.clip(mk - mk * mk, 0.0, None) + 1e-8)
    ss = jnp.sqrt(jnp.clip(qs - ms * ms, 0.0, None) + 1e-8)
    h = jnp.concatenate([mt, mk, ms, xt, xk, am, st, sk, ss], axis=1)
    h1 = _leaky(jnp.dot(h, w1[...], preferred_element_type=jnp.float32) + b1[...])
    h2 = _leaky(jnp.dot(h1, w2[...], preferred_element_type=jnp.float32) + b2[...])
    out[...] = jnp.sum(h2 * w3[...], axis=1, keepdims=True) + b3[...]


def _tc_mlp(ht, hk, hs, hm, w1, b1, w2, b2, w3row, b3):
  return pl.pallas_call(
      _tc_mlp_body,
      grid=(_NW,),
      in_specs=[
          pl.BlockSpec((1, _G, _NT), lambda i: (i, 0, 0)),
          pl.BlockSpec((1, _G, _NK), lambda i: (i, 0, 0)),
          pl.BlockSpec((1, _G, 4), lambda i: (i, 0, 0)),
          pl.BlockSpec((1, _G, 2), lambda i: (i, 0, 0)),
          pl.BlockSpec((777, 256), lambda i: (0, 0)),
          pl.BlockSpec((1, 256), lambda i: (0, 0)),
          pl.BlockSpec((256, 256), lambda i: (0, 0)),
          pl.BlockSpec((1, 256), lambda i: (0, 0)),
          pl.BlockSpec((1, 256), lambda i: (0, 0)),
          pl.BlockSpec((1, 1), lambda i: (0, 0)),
      ],
      out_specs=pl.BlockSpec((_G, 1), lambda i: (0, 0)),
      out_shape=jax.ShapeDtypeStruct((_G, 1), jnp.float32),
      scratch_shapes=[
          pltpu.VMEM((_G, _NT), jnp.float32),
          pltpu.VMEM((_G, _NK), jnp.float32),
          pltpu.VMEM((_G, 4), jnp.float32),
          pltpu.VMEM((_G, 2), jnp.float32),
      ],
      compiler_params=pltpu.CompilerParams(
          dimension_semantics=("arbitrary",)),
  )(ht, hk, hs, hm, w1, b1, w2, b2, w3row, b3)


def kernel(x_type, x_tok, x_small, batch, W1, b1, W2, b2, W3, b3):
  bat = batch.astype(jnp.int32)
  typ = x_type.astype(jnp.int32)
  tok = x_tok.astype(jnp.int32)
  xs = x_small.astype(jnp.float32)
  ht, hk, hs, hm = _sc_pool(bat, typ, tok, xs[:, 0], xs[:, 1])
  out = _tc_mlp(
      ht.reshape(_NW, _G, _NT),
      hk.reshape(_NW, _G, _NK),
      hs.reshape(_NW, _G, 4),
      hm.reshape(_NW, _G, 2),
      W1,
      b1.reshape(1, 256),
      W2,
      b2.reshape(1, 256),
      W3.reshape(1, 256),
      b3.reshape(1, 1),
  )
  return out.reshape(-1)


# speedup vs baseline: 22.7135x; 1.4342x over previous
import functools

import numpy as np

import jax
import jax.numpy as jnp
from jax import lax
from jax.experimental import pallas as pl
from jax.experimental.pallas import tpu as pltpu
from jax.experimental.pallas import tpu_sc as plsc

_N = 160000
_G = 512
_NT = 128
_NK = 129
_NH = _NT // 2
_NW = 32
_CHUNK = _N // _NW
_SBUF = 5008
_ITER = _SBUF // 16
_NEG = float("-inf")


def _perm_block(base):
  evens = list(range(0, _NT, 2))
  odds = list(range(1, _NT, 2))
  rows = [base + i for i in evens] + [base + i for i in odds]
  rows += [base + _NT + i for i in evens] + [base + _NT + i for i in odds]
  rows += [base + 256, base + 257, base + 258]
  return rows

_PERM = np.array(_perm_block(0) + _perm_block(259) + _perm_block(518),
                 dtype=np.int32)


def _take(x, idx):
  return x.at[idx].get(mode="promise_in_bounds")


def _sc_pool_body(bat_h, typ_h, tok_h, xs0_h, xs1_h,
                  out_t, out_k, out_sm,
                  b_v, t_v, s0_v, s1_v, hist, smacc):
  cid = lax.axis_index("c")
  sid = lax.axis_index("s")
  wid = sid * 2 + cid
  base = wid * _CHUNK
  pltpu.sync_copy(bat_h.at[pl.ds(base, _CHUNK)], b_v.at[pl.ds(0, _CHUNK)])
  pltpu.sync_copy(typ_h.at[pl.ds(base, _CHUNK)], t_v.at[pl.ds(0, _CHUNK)])
  pltpu.sync_copy(xs0_h.at[pl.ds(base, _CHUNK)], s0_v.at[pl.ds(0, _CHUNK)])
  pltpu.sync_copy(xs1_h.at[pl.ds(base, _CHUNK)], s1_v.at[pl.ds(0, _CHUNK)])

  lanes = lax.iota(jnp.int32, 16)
  izero16 = jnp.zeros((16,), jnp.int32)

  sm_init = jnp.where((lanes == 4) | (lanes == 5), _NEG, 0.0).astype(jnp.float32)

  def _zero_hist(j, c):
    for q in range(8):
      hist[pl.ds(j * _NT + q * 16, 16)] = izero16
    return c

  def _init_smacc(j, c):
    smacc[pl.ds(j * 16, 16)] = sm_init
    return c

  lax.fori_loop(0, _G, _zero_hist, 0)
  lax.fori_loop(0, _G, _init_smacc, 0)

  def _pass_a(i, c):
    off = i * 16
    valid = (off + lanes) < _CHUNK
    b = jnp.where(valid, b_v[pl.ds(off, 16)], -1)
    t = t_v[pl.ds(off, 16)]
    cnt, lastm = plsc.scan_count(b * _NT + t, mask=valid)
    odd = (t & 1) == 1
    col = b * _NT + lax.shift_right_logical(t, 1)
    plsc.addupdate_scatter(hist, [col], cnt, mask=lastm & jnp.logical_not(odd))
    plsc.addupdate_scatter(hist, [col], lax.shift_left(cnt, 16),
                           mask=lastm & odd)
    return c

  lax.fori_loop(0, _ITER, _pass_a, 0)
  pltpu.sync_copy(hist, out_t.at[wid])
  lax.fori_loop(0, _G, _zero_hist, 0)
  pltpu.sync_copy(tok_h.at[pl.ds(base, _CHUNK)], t_v.at[pl.ds(0, _CHUNK)])

  def _pass_b(i, c):
    off = i * 16
    valid = (off + lanes) < _CHUNK
    b = jnp.where(valid, b_v[pl.ds(off, 16)], -1)
    k = jnp.clip(t_v[pl.ds(off, 16)], 0, _NK - 1)
    cnt, lastm = plsc.scan_count(b * 256 + k, mask=valid)
    inb = lastm & (k < _NT)
    odd = (k & 1) == 1
    col = b * _NT + lax.shift_right_logical(k, 1)
    plsc.addupdate_scatter(hist, [col], cnt, mask=inb & jnp.logical_not(odd))
    plsc.addupdate_scatter(hist, [col], lax.shift_left(cnt, 16),
                           mask=inb & odd)

    v0 = jnp.where(valid, s0_v[pl.ds(off, 16)], 0.0)
    v1 = jnp.where(valid, s1_v[pl.ds(off, 16)], 0.0)
    s0 = v0
    s1 = v1
    q0 = v0 * v0
    q1 = v1 * v1
    m0 = jnp.where(valid, v0, _NEG)
    m1 = jnp.where(valid, v1, _NEG)
    for d in (1, 2, 4, 8):
      sidx = jnp.maximum(lanes - d, 0)
      bd = _take(b, sidx)
      same = (bd == b) & (lanes >= d)
      s0 = s0 + jnp.where(same, _take(s0, sidx), 0.0)
      s1 = s1 + jnp.where(same, _take(s1, sidx), 0.0)
      q0 = q0 + jnp.where(same, _take(q0, sidx), 0.0)
      q1 = q1 + jnp.where(same, _take(q1, sidx), 0.0)
      m0 = jnp.maximum(m0, jnp.where(same, _take(m0, sidx), _NEG))
      m1 = jnp.maximum(m1, jnp.where(same, _take(m1, sidx), _NEG))
    nb = _take(b, jnp.minimum(lanes + 1, 15))
    lastseg = ((b != nb) | (lanes == 15)) & valid
    sb = b * 16
    plsc.addupdate_scatter(smacc, [sb], s0, mask=lastseg)
    plsc.addupdate_scatter(smacc, [sb + 1], s1, mask=lastseg)
    plsc.addupdate_scatter(smacc, [sb + 2], q0, mask=lastseg)
    plsc.addupdate_scatter(smacc, [sb + 3], q1, mask=lastseg)
    cur0 = plsc.load_gather(smacc, [sb + 4], mask=lastseg)
    plsc.store_scatter(smacc, [sb + 4], jnp.maximum(cur0, m0), mask=lastseg)
    cur1 = plsc.load_gather(smacc, [sb + 5], mask=lastseg)
    plsc.store_scatter(smacc, [sb + 5], jnp.maximum(cur1, m1), mask=lastseg)
    return c

  lax.fori_loop(0, _ITER, _pass_b, 0)
  pltpu.sync_copy(hist, out_k.at[wid])
  pltpu.sync_copy(smacc, out_sm.at[wid])


_sc_pool = functools.partial(
    pl.kernel,
    out_type=[
        jax.ShapeDtypeStruct((_NW, _G * _NT), jnp.int32),
        jax.ShapeDtypeStruct((_NW, _G * _NT), jnp.int32),
        jax.ShapeDtypeStruct((_NW, _G * 16), jnp.float32),
    ],
    mesh=plsc.VectorSubcoreMesh(core_axis_name="c", subcore_axis_name="s"),
    compiler_params=pltpu.CompilerParams(needs_layout_passes=False),
    scratch_types=[
        pltpu.VMEM((_SBUF,), jnp.int32),
        pltpu.VMEM((_SBUF,), jnp.int32),
        pltpu.VMEM((_SBUF,), jnp.float32),
        pltpu.VMEM((_SBUF,), jnp.float32),
        pltpu.VMEM((_G * _NT,), jnp.int32),
        pltpu.VMEM((_G * 16,), jnp.float32),
    ],
)(_sc_pool_body)


def _leaky(v):
  return jnp.where(v > 0, v, 0.01 * v)


def _unpack(w):
  lo = jnp.bitwise_and(w, 0xFFFF).astype(jnp.float32)
  hi = lax.shift_right_logical(w, 16).astype(jnp.float32)
  return jnp.concatenate([lo, hi], axis=1)


def _tc_mlp_body(ht, hk, sm, w1, b1, w2, b2, w3, b3, out,
                 acc_t, acc_k, acc_s, acc_m):
  i = pl.program_id(0)

  @pl.when(i == 0)
  def _():
    acc_t[...] = jnp.zeros_like(acc_t)
    acc_k[...] = jnp.zeros_like(acc_k)
    acc_s[...] = jnp.zeros_like(acc_s)
    acc_m[...] = jnp.full_like(acc_m, _NEG)

  acc_t[...] += _unpack(ht[0, :, 0:_NH])
  acc_k[...] += _unpack(hk[0, :, 0:_NH])
  acc_s[...] += sm[0, :, 0:4]
  acc_m[...] = jnp.maximum(acc_m[...], sm[0, :, 4:6])

  @pl.when(i == _NW - 1)
  def _():
    at = acc_t[...]
    ak = acc_k[...]
    sums = acc_s[...]
    maxs = acc_m[...]
    cnt = jnp.sum(at, axis=1, keepdims=True)
    k128 = cnt - jnp.sum(ak, axis=1, keepdims=True)
    cntc = jnp.maximum(cnt, 1.0)
    mt = at / cntc
    mk = ak / cntc
    mk1 = k128 / cntc
    ms = sums[:, 0:2] / cntc
    qs = sums[:, 2:4] / cntc
    empty = cnt <= 0.0
    xt = jnp.where(empty, _NEG, (at > 0).astype(jnp.float32))
    xk = jnp.where(empty, _NEG, (ak > 0).astype(jnp.float32))
    xk1 = jnp.where(empty, _NEG, (k128 > 0).astype(jnp.float32))
    st = jnp.sqrt(jnp.clip(mt - mt * mt, 0.0, None) + 1e-8)
    sk = jnp.sqrt(jnp.clip(mk - mk * mk, 0.0, None) + 1e-8)
    sk1 = jnp.sqrt(jnp.clip(mk1 - mk1 * mk1, 0.0, None) + 1e-8)
    ss = jnp.sqrt(jnp.clip(qs - ms * ms, 0.0, None) + 1e-8)
    h = jnp.concatenate(
        [mt, mk, mk1, ms, xt, xk, xk1, maxs, st, sk, sk1, ss], axis=1)
    h1 = _leaky(jnp.dot(h, w1[...], preferred_element_type=jnp.float32) + b1[...])
    h2 = _leaky(jnp.dot(h1, w2[...], preferred_element_type=jnp.float32) + b2[...])
    out[...] = jnp.sum(h2 * w3[...], axis=1, keepdims=True) + b3[...]


def _tc_mlp(ht, hk, sm, w1p, b1, w2, b2, w3row, b3):
  return pl.pallas_call(
      _tc_mlp_body,
      grid=(_NW,),
      in_specs=[
          pl.BlockSpec((1, _G, _NT), lambda i: (i, 0, 0)),
          pl.BlockSpec((1, _G, _NT), lambda i: (i, 0, 0)),
          pl.BlockSpec((1, _G, 16), lambda i: (i, 0, 0)),
          pl.BlockSpec((777, 256), lambda i: (0, 0)),
          pl.BlockSpec((1, 256), lambda i: (0, 0)),
          pl.BlockSpec((256, 256), lambda i: (0, 0)),
          pl.BlockSpec((1, 256), lambda i: (0, 0)),
          pl.BlockSpec((1, 256), lambda i: (0, 0)),
          pl.BlockSpec((1, 1), lambda i: (0, 0)),
      ],
      out_specs=pl.BlockSpec((_G, 1), lambda i: (0, 0)),
      out_shape=jax.ShapeDtypeStruct((_G, 1), jnp.float32),
      scratch_shapes=[
          pltpu.VMEM((_G, _NT), jnp.float32),
          pltpu.VMEM((_G, _NT), jnp.float32),
          pltpu.VMEM((_G, 4), jnp.float32),
          pltpu.VMEM((_G, 2), jnp.float32),
      ],
      compiler_params=pltpu.CompilerParams(
          dimension_semantics=("arbitrary",)),
  )(ht, hk, sm, w1p, b1, w2, b2, w3row, b3)


def kernel(x_type, x_tok, x_small, batch, W1, b1, W2, b2, W3, b3):
  bat = batch.astype(jnp.int32)
  typ = x_type.astype(jnp.int32)
  tok = x_tok.astype(jnp.int32)
  xs = x_small.astype(jnp.float32)
  ht, hk, sm = _sc_pool(bat, typ, tok, xs[:, 0], xs[:, 1])
  out = _tc_mlp(
      ht.reshape(_NW, _G, _NT),
      hk.reshape(_NW, _G, _NT),
      sm.reshape(_NW, _G, 16),
      W1[_PERM],
      b1.reshape(1, 256),
      W2,
      b2.reshape(1, 256),
      W3.reshape(1, 256),
      b3.reshape(1, 1),
  )
  return out.reshape(-1)


# speedup vs baseline: 28.5645x; 1.2576x over previous
import functools

import numpy as np

import jax
import jax.numpy as jnp
from jax import lax
from jax.experimental import pallas as pl
from jax.experimental.pallas import tpu as pltpu
from jax.experimental.pallas import tpu_sc as plsc

_N = 160000
_G = 512
_NT = 128
_NK = 129
_NH = _NT // 2
_NW = 32
_CHUNK = _N // _NW
_SBUF = 5008
_ITER = _SBUF // 16
_HW = _G * _NH
_NEG = float("-inf")


_PERM_BIG = np.concatenate([np.arange(b, b + _NT) for b in
                            (0, 128, 259, 387, 518, 646)]).astype(np.int32)
_B128 = np.array([256, 515, 774], dtype=np.int32)
_SMALL = np.array([257, 258, 516, 517, 775, 776], dtype=np.int32)


def _take(x, idx):
  return x.at[idx].get(mode="promise_in_bounds")


def _sc_pool_body(bat_h, typ_h, tok_h, xs0_h, xs1_h,
                  out_t, out_k, out_sm,
                  b_v, t_v, s0_v, s1_v, hist, smacc):
  cid = lax.axis_index("c")
  sid = lax.axis_index("s")
  wid = sid * 2 + cid
  base = wid * _CHUNK
  pltpu.sync_copy(bat_h.at[pl.ds(base, _CHUNK)], b_v.at[pl.ds(0, _CHUNK)])
  pltpu.sync_copy(typ_h.at[pl.ds(base, _CHUNK)], t_v.at[pl.ds(0, _CHUNK)])
  pltpu.sync_copy(xs0_h.at[pl.ds(base, _CHUNK)], s0_v.at[pl.ds(0, _CHUNK)])
  pltpu.sync_copy(xs1_h.at[pl.ds(base, _CHUNK)], s1_v.at[pl.ds(0, _CHUNK)])

  lanes = lax.iota(jnp.int32, 16)
  izero16 = jnp.zeros((16,), jnp.int32)
  zero16 = jnp.zeros((16,), jnp.float32)
  ninf16 = jnp.full((16,), _NEG, jnp.float32)

  def _zero_hist(j, c):
    for q in range(8):
      hist[pl.ds(j * 128 + q * 16, 16)] = izero16
    return c

  def _init_smacc(j, c):
    smacc[pl.ds(j * 16, 16)] = jnp.where((j >= 128) & (j < 192), ninf16, zero16)
    return c

  lax.fori_loop(0, _HW // 128, _zero_hist, 0)
  lax.fori_loop(0, (_G * 16) // 16, _init_smacc, 0)

  def _pass_a(i, c):
    off = i * 16
    valid = (off + lanes) < _CHUNK
    b = jnp.where(valid, b_v[pl.ds(off, 16)], -1)
    t = t_v[pl.ds(off, 16)]
    cnt, lastm = plsc.scan_count(b * _NT + t, mask=valid)
    odd = (b & 1) == 1
    col = lax.shift_right_logical(b, 1) * _NT + t
    plsc.addupdate_scatter(hist, [col], cnt, mask=lastm & jnp.logical_not(odd))
    plsc.addupdate_scatter(hist, [col], lax.shift_left(cnt, 16),
                           mask=lastm & odd)
    return c

  lax.fori_loop(0, _ITER, _pass_a, 0)
  pltpu.sync_copy(hist, out_t.at[wid])
  lax.fori_loop(0, _HW // 128, _zero_hist, 0)
  pltpu.sync_copy(tok_h.at[pl.ds(base, _CHUNK)], t_v.at[pl.ds(0, _CHUNK)])

  def _pass_b(i, c):
    off = i * 16
    valid = (off + lanes) < _CHUNK
    b = jnp.where(valid, b_v[pl.ds(off, 16)], -1)
    k = jnp.clip(t_v[pl.ds(off, 16)], 0, _NK - 1)
    cnt, lastm = plsc.scan_count(b * 256 + k, mask=valid)
    inb = lastm & (k < _NT)
    odd = (b & 1) == 1
    col = lax.shift_right_logical(b, 1) * _NT + k
    plsc.addupdate_scatter(hist, [col], cnt, mask=inb & jnp.logical_not(odd))
    plsc.addupdate_scatter(hist, [col], lax.shift_left(cnt, 16),
                           mask=inb & odd)

    v0 = jnp.where(valid, s0_v[pl.ds(off, 16)], 0.0)
    v1 = jnp.where(valid, s1_v[pl.ds(off, 16)], 0.0)
    s0 = v0
    s1 = v1
    q0 = v0 * v0
    q1 = v1 * v1
    c1 = jnp.where(valid, 1.0, 0.0)
    m0 = jnp.where(valid, v0, _NEG)
    m1 = jnp.where(valid, v1, _NEG)
    for d in (1, 2, 4, 8):
      sidx = jnp.maximum(lanes - d, 0)
      bd = _take(b, sidx)
      same = (bd == b) & (lanes >= d)
      c1 = c1 + jnp.where(same, _take(c1, sidx), 0.0)
      s0 = s0 + jnp.where(same, _take(s0, sidx), 0.0)
      s1 = s1 + jnp.where(same, _take(s1, sidx), 0.0)
      q0 = q0 + jnp.where(same, _take(q0, sidx), 0.0)
      q1 = q1 + jnp.where(same, _take(q1, sidx), 0.0)
      m0 = jnp.maximum(m0, jnp.where(same, _take(m0, sidx), _NEG))
      m1 = jnp.maximum(m1, jnp.where(same, _take(m1, sidx), _NEG))
    nb = _take(b, jnp.minimum(lanes + 1, 15))
    lastseg = ((b != nb) | (lanes == 15)) & valid
    plsc.addupdate_scatter(smacc, [b], s0, mask=lastseg)
    plsc.addupdate_scatter(smacc, [b + 512], s1, mask=lastseg)
    plsc.addupdate_scatter(smacc, [b + 1024], q0, mask=lastseg)
    plsc.addupdate_scatter(smacc, [b + 1536], q1, mask=lastseg)
    plsc.addupdate_scatter(smacc, [b + 3072], c1, mask=lastseg)
    cur0 = plsc.load_gather(smacc, [b + 2048], mask=lastseg)
    plsc.store_scatter(smacc, [b + 2048], jnp.maximum(cur0, m0), mask=lastseg)
    cur1 = plsc.load_gather(smacc, [b + 2560], mask=lastseg)
    plsc.store_scatter(smacc, [b + 2560], jnp.maximum(cur1, m1), mask=lastseg)
    return c

  lax.fori_loop(0, _ITER, _pass_b, 0)
  pltpu.sync_copy(hist, out_k.at[wid])
  pltpu.sync_copy(smacc, out_sm.at[wid])


_sc_pool = functools.partial(
    pl.kernel,
    out_type=[
        jax.ShapeDtypeStruct((_NW, _HW), jnp.int32),
        jax.ShapeDtypeStruct((_NW, _HW), jnp.int32),
        jax.ShapeDtypeStruct((_NW, _G * 16), jnp.float32),
    ],
    mesh=plsc.VectorSubcoreMesh(core_axis_name="c", subcore_axis_name="s"),
    compiler_params=pltpu.CompilerParams(needs_layout_passes=False),
    scratch_types=[
        pltpu.VMEM((_SBUF,), jnp.int32),
        pltpu.VMEM((_SBUF,), jnp.int32),
        pltpu.VMEM((_SBUF,), jnp.float32),
        pltpu.VMEM((_SBUF,), jnp.float32),
        pltpu.VMEM((_HW,), jnp.int32),
        pltpu.VMEM((_G * 16,), jnp.float32),
    ],
)(_sc_pool_body)


def _leaky(v):
  return jnp.where(v > 0, v, 0.01 * v)


def _tc_mlp_body(ht, hk, sm, w1b, w1c, w1s, b1, w2, b2, w3, b3, out,
                 alo_t, ahi_t, alo_k, ahi_k, acc_s, acc_m, acc_c):
  i = pl.program_id(0)

  @pl.when(i == 0)
  def _():
    alo_t[...] = jnp.zeros_like(alo_t)
    ahi_t[...] = jnp.zeros_like(ahi_t)
    alo_k[...] = jnp.zeros_like(alo_k)
    ahi_k[...] = jnp.zeros_like(ahi_k)
    acc_s[...] = jnp.zeros_like(acc_s)
    acc_m[...] = jnp.full_like(acc_m, _NEG)
    acc_c[...] = jnp.zeros_like(acc_c)

  wt = ht[0]
  wk = hk[0]
  alo_t[...] += jnp.bitwise_and(wt, 0xFFFF).astype(jnp.float32)
  ahi_t[...] += lax.shift_right_logical(wt, 16).astype(jnp.float32)
  alo_k[...] += jnp.bitwise_and(wk, 0xFFFF).astype(jnp.float32)
  ahi_k[...] += lax.shift_right_logical(wk, 16).astype(jnp.float32)
  acc_s[...] += sm[0, 0:4, :]
  acc_m[...] = jnp.maximum(acc_m[...], sm[0, 4:6, :])
  acc_c[...] += sm[0, 6:7, :]

  @pl.when(i == _NW - 1)
  def _():
    at = jnp.stack([alo_t[...], ahi_t[...]], axis=1).reshape(_G, _NT)
    ak = jnp.stack([alo_k[...], ahi_k[...]], axis=1).reshape(_G, _NT)
    cnt = jnp.sum(at, axis=1, keepdims=True)
    cntc = jnp.maximum(cnt, 1.0)
    empty = cnt <= 0.0
    mt = at / cntc
    mk = ak / cntc
    xt = jnp.where(empty, _NEG, (at > 0).astype(jnp.float32))
    xk = jnp.where(empty, _NEG, (ak > 0).astype(jnp.float32))
    st = jnp.sqrt(jnp.clip(mt - mt * mt, 0.0, None) + 1e-8)
    sk = jnp.sqrt(jnp.clip(mk - mk * mk, 0.0, None) + 1e-8)
    k128 = cnt - jnp.sum(ak, axis=1, keepdims=True)
    mk1 = k128 / cntc
    xk1 = jnp.where(empty, _NEG, (k128 > 0).astype(jnp.float32))
    sk1 = jnp.sqrt(jnp.clip(mk1 - mk1 * mk1, 0.0, None) + 1e-8)

    sums = acc_s[...]
    maxs = acc_m[...]
    cnt_r = jnp.maximum(acc_c[...], 1.0)
    ms = sums[0:2, :] / cnt_r
    qs = sums[2:4, :] / cnt_r
    ss = jnp.sqrt(jnp.clip(qs - ms * ms, 0.0, None) + 1e-8)
    small_f = jnp.concatenate([ms, maxs, ss], axis=0)

    hbig = jnp.concatenate([mt, mk, xt, xk, st, sk], axis=1)
    hb = jnp.concatenate([mk1, xk1, sk1], axis=1)
    z1 = jnp.dot(hbig, w1b[...], preferred_element_type=jnp.float32)
    z1 += jnp.dot(hb, w1c[...], preferred_element_type=jnp.float32)
    z1 += lax.dot_general(small_f, w1s[...], (((0,), (0,)), ((), ())),
                          preferred_element_type=jnp.float32)
    h1 = _leaky(z1 + b1[...])
    h2 = _leaky(jnp.dot(h1, w2[...], preferred_element_type=jnp.float32) + b2[...])
    out[...] = jnp.sum(h2 * w3[...], axis=1, keepdims=True) + b3[...]


def _tc_mlp(ht, hk, sm, w1b, w1c, w1s, b1, w2, b2, w3row, b3):
  return pl.pallas_call(
      _tc_mlp_body,
      grid=(_NW,),
      in_specs=[
          pl.BlockSpec((1, _G // 2, _NT), lambda i: (i, 0, 0)),
          pl.BlockSpec((1, _G // 2, _NT), lambda i: (i, 0, 0)),
          pl.BlockSpec((1, 16, _G), lambda i: (i, 0, 0)),
          pl.BlockSpec((768, 256), lambda i: (0, 0)),
          pl.BlockSpec((3, 256), lambda i: (0, 0)),
          pl.BlockSpec((6, 256), lambda i: (0, 0)),
          pl.BlockSpec((1, 256), lambda i: (0, 0)),
          pl.BlockSpec((256, 256), lambda i: (0, 0)),
          pl.BlockSpec((1, 256), lambda i: (0, 0)),
          pl.BlockSpec((1, 256), lambda i: (0, 0)),
          pl.BlockSpec((1, 1), lambda i: (0, 0)),
      ],
      out_specs=pl.BlockSpec((_G, 1), lambda i: (0, 0)),
      out_shape=jax.ShapeDtypeStruct((_G, 1), jnp.float32),
      scratch_shapes=[
          pltpu.VMEM((_G // 2, _NT), jnp.float32),
          pltpu.VMEM((_G // 2, _NT), jnp.float32),
          pltpu.VMEM((_G // 2, _NT), jnp.float32),
          pltpu.VMEM((_G // 2, _NT), jnp.float32),
          pltpu.VMEM((4, _G), jnp.float32),
          pltpu.VMEM((2, _G), jnp.float32),
          pltpu.VMEM((1, _G), jnp.float32),
      ],
      compiler_params=pltpu.CompilerParams(
          dimension_semantics=("arbitrary",)),
  )(ht, hk, sm, w1b, w1c, w1s, b1, w2, b2, w3row, b3)


def kernel(x_type, x_tok, x_small, batch, W1, b1, W2, b2, W3, b3):
  bat = batch.astype(jnp.int32)
  typ = x_type.astype(jnp.int32)
  tok = x_tok.astype(jnp.int32)
  xs = x_small.astype(jnp.float32)
  ht, hk, sm = _sc_pool(bat, typ, tok, xs[:, 0], xs[:, 1])
  out = _tc_mlp(
      ht.reshape(_NW, _G // 2, _NT),
      hk.reshape(_NW, _G // 2, _NT),
      sm.reshape(_NW, 16, _G),
      W1[_PERM_BIG],
      W1[_B128],
      W1[_SMALL],
      b1.reshape(1, 256),
      W2,
      b2.reshape(1, 256),
      W3.reshape(1, 256),
      b3.reshape(1, 1),
  )
  return out.reshape(-1)
